# baseline (speedup 1.0000x reference)
import functools

import jax
import jax.numpy as jnp
from jax.experimental import pallas as pl

_N = 512
_H = 200


def _dot(a, b):
    return jax.lax.dot_general(a, b, (((1,), (0,)), ((), ())),
                               preferred_element_type=jnp.float32)


def _dot_t(a, b):
    return jax.lax.dot_general(a, b, (((0,), (0,)), ((), ())),
                               preferred_element_type=jnp.float32)


def _dot_wt(a, w):
    return jax.lax.dot_general(a, w, (((1,), (1,)), ((), ())),
                               preferred_element_type=jnp.float32)


def _fused_body(adj_ref, gin_ref, keb_ref, wi0_ref, w200_ref, b_ref,
                wm_ref, out_ref, *, thr, pair):
    f32 = jnp.float32

    def gru(i, x, h):
        wi = wi0_ref[...] if i == 0 else w200_ref[i - 1]
        gxr = _dot_wt(x, wi[0:_H, :]) + b_ref[i, 0:1, :]
        gxz = _dot_wt(x, wi[_H:2 * _H, :]) + b_ref[i, 1:2, :]
        gxn = _dot_wt(x, wi[2 * _H:, :]) + b_ref[i, 2:3, :]
        if h is None:
            ghr = b_ref[6 + i, 0:1, :]
            ghz = b_ref[6 + i, 1:2, :]
            ghn = jnp.broadcast_to(b_ref[6 + i, 2:3, :], gxn.shape)
        else:
            wh = w200_ref[5 + i]
            ghr = _dot_wt(h, wh[0:_H, :]) + b_ref[6 + i, 0:1, :]
            ghz = _dot_wt(h, wh[_H:2 * _H, :]) + b_ref[6 + i, 1:2, :]
            ghn = _dot_wt(h, wh[2 * _H:, :]) + b_ref[6 + i, 2:3, :]
        r = jax.nn.sigmoid(gxr + ghr)
        z = jax.nn.sigmoid(gxz + ghz)
        n = jnp.tanh(gxn + r * ghn)
        if h is None:
            return (1.0 - z) * n
        return (1.0 - z) * n + z * h

    def graph_chain(g):
        a = (adj_ref[g] < thr).astype(f32)
        deg_in = jnp.maximum(jnp.sum(a, axis=1, keepdims=True), 1.0)
        deg_out = jnp.maximum(jnp.sum(a, axis=0, keepdims=True), 1.0)
        ar = a / deg_in
        ac = a / deg_out

        h = gru(0, _dot(ar, gin_ref[g]), None)
        vo0 = h[0:3, :]
        h = gru(1, _dot_t(ac, h), h)
        h = gru(2, _dot(ar, h), h)
        vo1 = h[0:3, :]
        h = gru(3, _dot_t(ac, h), h)
        vo2 = gru(4, _dot(ar[0:3, :], h), h[0:3, :])

        hv = gru(5, vo0, None)
        hv = gru(5, vo1, hv)
        hv = gru(5, vo2, hv)

        hg = jnp.concatenate([hv[0:1, :], hv[1:2, :], hv[2:3, :]], axis=1)
        enc = _dot_wt(hg, wm_ref[...])
        d = enc - keb_ref[g]
        return jnp.sum(d * d)

    loss = graph_chain(0)
    for g in range(1, pair):
        loss = loss + graph_chain(g)
    loss = loss.reshape(1, 1)

    b = pl.program_id(0)

    @pl.when(b == 0)
    def _():
        out_ref[...] = loss

    @pl.when(b != 0)
    def _():
        out_ref[...] += loss


def kernel(g_in, g_adj, batch_size, kernel_embeddings, reg_solutions, params):
    del reg_solutions
    b, n, vt = g_in.shape
    thr = 16.0 / n
    pair = 2 if b % 2 == 0 else 1

    grus = [params["fw"][0], params["bw"][0], params["fw"][1],
            params["bw"][1], params["fw"][2], params["var"]]
    wi0 = grus[0]["Wi"]
    w200 = jnp.stack([p["Wi"] for p in grus[1:]] +
                     [p["Wh"] for p in grus])
    biases = jnp.stack([p["bi"] for p in grus] +
                       [p["bh"] for p in grus]).reshape(12, 3, _H)
    z = params["Wm"].shape[0]
    keb = (kernel_embeddings - params["bm"][None, :]).reshape(b, 1, z)

    full = lambda arr: pl.BlockSpec(arr.shape, lambda i: (0,) * arr.ndim)

    out = pl.pallas_call(
        functools.partial(_fused_body, thr=thr, pair=pair),
        grid=(b // pair,),
        in_specs=[
            pl.BlockSpec((pair, n, n), lambda i: (i, 0, 0)),
            pl.BlockSpec((pair, n, vt), lambda i: (i, 0, 0)),
            pl.BlockSpec((pair, 1, z), lambda i: (i, 0, 0)),
            full(wi0),
            full(w200),
            full(biases),
            full(params["Wm"]),
        ],
        out_specs=pl.BlockSpec((1, 1), lambda i: (0, 0)),
        out_shape=jax.ShapeDtypeStruct((1, 1), jnp.float32),
    )(g_adj, g_in, keb, wi0, w200, biases, params["Wm"])
    return out[0, 0]
